# Initial kernel scaffold; baseline (speedup 1.0000x reference)
import jax
import jax.numpy as jnp
from jax.experimental import pallas as pl


def kernel(decoder_input, encoder_output, params):
    raise NotImplementedError("write your pallas kernel here")


# speedup vs baseline: 6.3787x; 6.3787x over previous
import functools
import numpy as np
import jax
import jax.numpy as jnp
from jax.experimental import pallas as pl
from jax.experimental.pallas import tpu as pltpu

_B = 2
_S = 8192
_D = 768
_H = 12
_DK = 64
_DV = 64
_EXP = 4
_BKT = 64
_NC = _S // _BKT
_NHASH = _NC
_SB = 512
_CPB = 16
_NCB = _NC // _CPB


def _rot_const(seed):
    rng = np.random.default_rng(seed)
    nb = _S // _BKT
    r = rng.standard_normal((1, _H, _DK, nb // 2))
    return jnp.asarray(r[0], dtype=jnp.float32)


def _qkv_hash_body(xq_ref, xkv_ref, wq_ref, wk_ref, wv_ref, rot_ref,
                   q_ref, k_ref, v_ref, bq_ref, bk_ref):
    xq = xq_ref[0]
    xkv = xkv_ref[0]
    q = jnp.dot(xq, wq_ref[...], preferred_element_type=jnp.float32)
    k = jnp.dot(xkv, wk_ref[...], preferred_element_type=jnp.float32)
    v = jnp.dot(xkv, wv_ref[...], preferred_element_type=jnp.float32)
    q_ref[0] = q
    k_ref[0] = k
    v_ref[0] = v

    def buckets(mat):
        cols = []
        for h in range(_H):
            mh = mat[:, h * _DK:(h + 1) * _DK]
            rq = jnp.dot(mh, rot_ref[h], preferred_element_type=jnp.float32)
            sc = jnp.concatenate([rq, -rq], axis=-1)
            m = jnp.max(sc, axis=-1, keepdims=True)
            lane = jax.lax.broadcasted_iota(jnp.int32, sc.shape, 1)
            idx = jnp.min(jnp.where(sc >= m, lane, _NHASH), axis=-1)
            cols.append(idx.reshape(_SB, 1))
        cols.append(jnp.zeros((_SB, 128 - _H), jnp.int32))
        return jnp.concatenate(cols, axis=-1)

    bq_ref[0] = buckets(q)
    bk_ref[0] = buckets(k)


def _qkv_hash(xq, xkv, wq, wk, wv, rot):
    nsb = _S // _SB
    grid = (_B, nsb)
    io_spec = pl.BlockSpec((1, _SB, _D), lambda b, s: (b, s, 0))
    w_spec = pl.BlockSpec((_D, _H * _DK), lambda b, s: (0, 0))
    b_spec = pl.BlockSpec((1, _SB, 128), lambda b, s: (b, s, 0))
    out = pl.pallas_call(
        _qkv_hash_body,
        grid=grid,
        in_specs=[io_spec, io_spec, w_spec, w_spec, w_spec,
                  pl.BlockSpec((_H, _DK, 64), lambda b, s: (0, 0, 0))],
        out_specs=[io_spec, io_spec, io_spec, b_spec, b_spec],
        out_shape=[
            jax.ShapeDtypeStruct((_B, _S, _D), jnp.float32),
            jax.ShapeDtypeStruct((_B, _S, _D), jnp.float32),
            jax.ShapeDtypeStruct((_B, _S, _D), jnp.float32),
            jax.ShapeDtypeStruct((_B, _S, 128), jnp.int32),
            jax.ShapeDtypeStruct((_B, _S, 128), jnp.int32),
        ],
    )(xq, xkv, wq, wk, wv, rot)
    return out


def _attn_body(qs_ref, ks_ref, ksp_ref, vs_ref, vsp_ref, o_ref):
    scale = 1.0 / np.sqrt(_DK).astype(np.float32)
    for i in range(_CPB):
        qc = qs_ref[0, 0, i * _BKT:(i + 1) * _BKT, :]
        kc = ks_ref[0, 0, i * _BKT:(i + 1) * _BKT, :]
        vc = vs_ref[0, 0, i * _BKT:(i + 1) * _BKT, :]
        if i == 0:
            kp = ksp_ref[0, 0, (_CPB - 1) * _BKT:, :]
            vp = vsp_ref[0, 0, (_CPB - 1) * _BKT:, :]
        else:
            kp = ks_ref[0, 0, (i - 1) * _BKT:i * _BKT, :]
            vp = vs_ref[0, 0, (i - 1) * _BKT:i * _BKT, :]
        ke = jnp.concatenate([kc, kp], axis=0)
        ve = jnp.concatenate([vc, vp], axis=0)
        s = jnp.dot(qc, ke.T, preferred_element_type=jnp.float32) * scale
        m = jnp.max(s, axis=-1, keepdims=True)
        e = jnp.exp(s - m)
        a = e / jnp.sum(e, axis=-1, keepdims=True)
        o_ref[0, 0, i * _BKT:(i + 1) * _BKT, :] = jnp.dot(
            a, ve, preferred_element_type=jnp.float32)


def _chunk_attn(qs, ks, vs):
    grid = (_B, _H, _NCB)
    blk = _CPB * _BKT
    spec = pl.BlockSpec((1, 1, blk, _DK), lambda b, h, c: (b, h, c, 0))
    prev = pl.BlockSpec((1, 1, blk, _DK),
                        lambda b, h, c: (b, h, (c + _NCB - 1) % _NCB, 0))
    return pl.pallas_call(
        _attn_body,
        grid=grid,
        in_specs=[spec, spec, prev, spec, prev],
        out_specs=spec,
        out_shape=jax.ShapeDtypeStruct((_B, _H, _S, _DV), jnp.float32),
    )(qs, ks, ks, vs, vs)


def _ln(x, g, b):
    m = jnp.mean(x, axis=-1, keepdims=True)
    xc = x - m
    v = jnp.mean(xc * xc, axis=-1, keepdims=True)
    return xc * jax.lax.rsqrt(v + 1e-6) * g + b


def _proj_ln_body(o_ref, x_ref, wo_ref, g_ref, b_ref, out_ref):
    o = jnp.dot(o_ref[0], wo_ref[...], preferred_element_type=jnp.float32)
    out_ref[0] = _ln(x_ref[0] + o, g_ref[...], b_ref[...])


def _proj_ln(o, x, wo, g, b):
    grid = (_B, _S // _SB)
    io_spec = pl.BlockSpec((1, _SB, _D), lambda bb, s: (bb, s, 0))
    return pl.pallas_call(
        _proj_ln_body,
        grid=grid,
        in_specs=[io_spec, io_spec,
                  pl.BlockSpec((_D, _D), lambda bb, s: (0, 0)),
                  pl.BlockSpec((_D,), lambda bb, s: (0,)),
                  pl.BlockSpec((_D,), lambda bb, s: (0,))],
        out_specs=io_spec,
        out_shape=jax.ShapeDtypeStruct((_B, _S, _D), jnp.float32),
    )(o, x, wo, g, b)


def _ffn_body(x_ref, w1_ref, b1_ref, w2_ref, b2_ref, g_ref, b_ref, out_ref):
    x = x_ref[0]
    h = jnp.dot(x, w1_ref[...], preferred_element_type=jnp.float32) + b1_ref[...]
    h = jnp.maximum(h, 0.0)
    y = jnp.dot(h, w2_ref[...], preferred_element_type=jnp.float32) + b2_ref[...]
    out_ref[0] = _ln(x + y, g_ref[...], b_ref[...])


def _ffn(x, p):
    grid = (_B, _S // _SB)
    io_spec = pl.BlockSpec((1, _SB, _D), lambda bb, s: (bb, s, 0))
    return pl.pallas_call(
        _ffn_body,
        grid=grid,
        in_specs=[io_spec,
                  pl.BlockSpec((_D, _EXP * _D), lambda bb, s: (0, 0)),
                  pl.BlockSpec((_EXP * _D,), lambda bb, s: (0,)),
                  pl.BlockSpec((_EXP * _D, _D), lambda bb, s: (0, 0)),
                  pl.BlockSpec((_D,), lambda bb, s: (0,)),
                  pl.BlockSpec((_D,), lambda bb, s: (0,)),
                  pl.BlockSpec((_D,), lambda bb, s: (0,))],
        out_specs=io_spec,
        out_shape=jax.ShapeDtypeStruct((_B, _S, _D), jnp.float32),
    )(x, p['W1'], p['b1'], p['W2'], p['b2'], p['ln_g'], p['ln_b'])


def _lsh_attn(qin, kvin, p, rot):
    q, k, v, bq, bk = _qkv_hash(qin, kvin, p['Wq'], p['Wk'], p['Wv'], rot)
    qh = q.reshape(_B, _S, _H, _DK).transpose(0, 2, 1, 3)
    kh = k.reshape(_B, _S, _H, _DK).transpose(0, 2, 1, 3)
    vh = v.reshape(_B, _S, _H, _DV).transpose(0, 2, 1, 3)
    bq = bq[:, :, :_H].transpose(0, 2, 1)
    bk = bk[:, :, :_H].transpose(0, 2, 1)
    qi = jnp.argsort(bq, axis=-1)
    ki = jnp.argsort(bk, axis=-1)
    qs = jnp.take_along_axis(qh, qi[..., None], 2)
    ks = jnp.take_along_axis(kh, ki[..., None], 2)
    vs = jnp.take_along_axis(vh, ki[..., None], 2)
    oc = _chunk_attn(qs, ks, vs)
    inv = jnp.argsort(qi, axis=-1)
    o = jnp.take_along_axis(oc, inv[..., None], 2)
    o = o.transpose(0, 2, 1, 3).reshape(_B, _S, _H * _DV)
    return _proj_ln(o, qin, p['Wo'], p['ln_g'], p['ln_b'])


def kernel(decoder_input, encoder_output, params):
    rot1 = _rot_const(1)
    rot2 = _rot_const(2)
    x = _lsh_attn(decoder_input, decoder_input, params['self'], rot1)
    x = _lsh_attn(x, encoder_output, params['enc'], rot2)
    return _ffn(x, params['ff'])
